# Initial kernel scaffold; baseline (speedup 1.0000x reference)
import jax
import jax.numpy as jnp
from jax.experimental import pallas as pl


def kernel(x_user, x_item, edge_index_buys, edge_index_rev, edge_label_index, snap, W1_buys, W1_rev, Wp1, bp1, a1_Wu, a1_bu, a1_qu, a1_Wi, a1_bi, a1_qi, W2_buys, W2_rev, Wp2, bp2, a2_Wu, a2_bu, a2_qu, a2_Wi, a2_bi, a2_qi, W_post, b_post):
    raise NotImplementedError("write your pallas kernel here")


# speedup vs baseline: 4.4217x; 4.4217x over previous
import functools

import jax
import jax.numpy as jnp
from jax import lax
from jax.experimental import pallas as pl
from jax.experimental.pallas import tpu as pltpu
from jax.experimental.pallas import tpu_sc as plsc

N = 10000
D = 128
E = 640000
L = 100000

NPAD = 10112
CHUNK = 128
TILES = 16
EPT = 40064
EPAD = EPT * TILES
CPT = EPT // CHUNK
ZROWS = NPAD // TILES

LPAD = 102400
LPW = LPAD // 32
LCPW = LPW // CHUNK

_HI = jax.lax.Precision.HIGHEST


HD = D // 2


def _seg_sum_sc(with_counts):
  mesh = plsc.VectorSubcoreMesh(core_axis_name="c", subcore_axis_name="s")
  out_type = [jax.ShapeDtypeStruct((NPAD, HD), jnp.float32)
              for _ in range(4)]
  if with_counts:
    out_type += [
        jax.ShapeDtypeStruct((NPAD, 16), jnp.float32),
        jax.ShapeDtypeStruct((NPAD, 16), jnp.float32),
    ]
  scratch = [
      pltpu.VMEM((CHUNK,), jnp.int32),
      pltpu.VMEM((CHUNK,), jnp.int32),
      pltpu.VMEM((CHUNK, HD), jnp.float32),
      pltpu.VMEM((CHUNK, 16), jnp.float32),
      pltpu.VMEM((ZROWS, HD), jnp.float32),
      pltpu.VMEM((ZROWS, 16), jnp.float32),
      pltpu.VMEM_SHARED((NPAD, HD), jnp.float32),
      pltpu.VMEM_SHARED((NPAD, 16), jnp.float32),
      pltpu.SemaphoreType.DMA,
  ]

  def body(xa_lo, xa_hi, xb_lo, xb_hi, src_a, dst_a, src_b, dst_b,
           z64, z16, ones_h, *refs):
    if with_counts:
      out_a_lo, out_a_hi, out_b_lo, out_b_hi, cnt_a, cnt_b = refs[:6]
      scr = refs[6:]
    else:
      out_a_lo, out_a_hi, out_b_lo, out_b_hi = refs[:4]
      scr = refs[4:]
    sidx, didx, rows, ones_v, stage, stage16, acc, cacc, sem = scr
    cid = lax.axis_index("c")
    sid = lax.axis_index("s")
    r0 = sid * ZROWS

    if with_counts:
      pltpu.sync_copy(ones_h, ones_v)

    for phase in range(2):
      do_cnt = with_counts and phase == 0
      tab_a = xa_lo if phase == 0 else xa_hi
      tab_b = xb_lo if phase == 0 else xb_hi
      out_a = out_a_lo if phase == 0 else out_a_hi
      out_b = out_b_lo if phase == 0 else out_b_hi

      pltpu.sync_copy(z64.at[pl.ds(r0, ZROWS)], stage)
      pltpu.sync_copy(stage, acc.at[pl.ds(r0, ZROWS)])
      if do_cnt:
        pltpu.sync_copy(z16.at[pl.ds(r0, ZROWS)], stage16)
        pltpu.sync_copy(stage16, cacc.at[pl.ds(r0, ZROWS)])
      plsc.subcore_barrier()

      def make_loop(src_h, dst_h, tab_h, count_now):
        def step(c, _):
          off = sid * EPT + c * CHUNK
          pltpu.sync_copy(src_h.at[pl.ds(off, CHUNK)], sidx)
          pltpu.sync_copy(dst_h.at[pl.ds(off, CHUNK)], didx)
          pltpu.async_copy(tab_h.at[sidx], rows, sem).wait()
          pltpu.sync_copy(rows, acc.at[didx], add=True)
          if count_now:
            pltpu.sync_copy(ones_v, cacc.at[didx], add=True)
          return 0
        return step

      @pl.when(cid == 0)
      def _():
        lax.fori_loop(0, CPT, make_loop(src_a, dst_a, tab_a, do_cnt), 0)

      @pl.when(cid == 1)
      def _():
        lax.fori_loop(0, CPT, make_loop(src_b, dst_b, tab_b, do_cnt), 0)

      plsc.subcore_barrier()

      pltpu.sync_copy(acc.at[pl.ds(r0, ZROWS)], stage)
      if do_cnt:
        pltpu.sync_copy(cacc.at[pl.ds(r0, ZROWS)], stage16)

      @pl.when(cid == 0)
      def _():
        pltpu.sync_copy(stage, out_a.at[pl.ds(r0, ZROWS)])
        if do_cnt:
          pltpu.sync_copy(stage16, cnt_a.at[pl.ds(r0, ZROWS)])

      @pl.when(cid == 1)
      def _():
        pltpu.sync_copy(stage, out_b.at[pl.ds(r0, ZROWS)])
        if do_cnt:
          pltpu.sync_copy(stage16, cnt_b.at[pl.ds(r0, ZROWS)])

      if phase == 0:
        plsc.subcore_barrier()

  return pl.kernel(
      body, out_type=out_type, mesh=mesh, scratch_types=scratch,
      compiler_params=pltpu.CompilerParams(use_tc_tiling_on_sc=False))


def _head_gather_sc():
  mesh = plsc.VectorSubcoreMesh(core_axis_name="c", subcore_axis_name="s")
  out_type = [
      jax.ShapeDtypeStruct((LPAD, D), jnp.float32),
      jax.ShapeDtypeStruct((LPAD, D), jnp.float32),
  ]
  scratch = [
      pltpu.VMEM((CHUNK,), jnp.int32),
      pltpu.VMEM((CHUNK, D), jnp.float32),
      pltpu.SemaphoreType.DMA,
  ]

  def body(u_tab, i_tab, ls, ld, hs, hd, idx, rows, sem):
    cid = lax.axis_index("c")
    sid = lax.axis_index("s")
    wid = sid * 2 + cid

    def step(c, _):
      off = wid * LPW + c * CHUNK
      pltpu.sync_copy(ls.at[pl.ds(off, CHUNK)], idx)
      pltpu.async_copy(u_tab.at[idx], rows, sem).wait()
      pltpu.sync_copy(rows, hs.at[pl.ds(off, CHUNK)])
      pltpu.sync_copy(ld.at[pl.ds(off, CHUNK)], idx)
      pltpu.async_copy(i_tab.at[idx], rows, sem).wait()
      pltpu.sync_copy(rows, hd.at[pl.ds(off, CHUNK)])
      return 0

    lax.fori_loop(0, LCPW, step, 0)

  return pl.kernel(body, out_type=out_type, mesh=mesh, scratch_types=scratch)


_DROWS = 632


def _tc_dense_body(si, ci, su, cu, wi, wua, wub, bu, mi, mu):
  inv_i = 1.0 / jnp.maximum(ci[:, 0:1], 1.0)
  mi[:] = jnp.dot(si[:] * inv_i, wi[:], precision=_HI)
  inv_u = 1.0 / jnp.maximum(cu[:, 0:1], 1.0)
  wu = jnp.dot(wua[:], wub[:], precision=_HI)
  mu[:] = jnp.dot(su[:] * inv_u, wu, precision=_HI) + bu[:]


def _tc_dense(si, ci, su, cu, wi, wua, wub, bu):
  grid = NPAD // _DROWS
  blk_nd = pl.BlockSpec((_DROWS, D), lambda i: (i, 0))
  blk_cnt = pl.BlockSpec((_DROWS, 16), lambda i: (i, 0))
  blk_w = pl.BlockSpec((D, D), lambda i: (0, 0))
  blk_b = pl.BlockSpec((1, D), lambda i: (0, 0))
  return pl.pallas_call(
      _tc_dense_body,
      grid=(grid,),
      in_specs=[blk_nd, blk_cnt, blk_nd, blk_cnt, blk_w, blk_w, blk_w, blk_b],
      out_specs=[blk_nd, blk_nd],
      out_shape=[
          jax.ShapeDtypeStruct((NPAD, D), jnp.float32),
          jax.ShapeDtypeStruct((NPAD, D), jnp.float32),
      ],
  )(si, ci, su, cu, wi, wua, wub, bu)


_HROWS = 2048


def _tc_head_body(hs, hd, wp, bp, out):
  w = jnp.sum(wp[:], axis=1)
  c = jnp.sum(bp[:])
  p = hs[:] * hd[:] * w[None, :]
  out[:] = jnp.sum(p, axis=1, keepdims=True) + c


def _tc_head(hs, hd, wp, bp):
  grid = LPAD // _HROWS
  blk = pl.BlockSpec((_HROWS, D), lambda i: (i, 0))
  return pl.pallas_call(
      _tc_head_body,
      grid=(grid,),
      in_specs=[blk, blk,
                pl.BlockSpec((D, 2), lambda i: (0, 0)),
                pl.BlockSpec((1, 2), lambda i: (0, 0))],
      out_specs=pl.BlockSpec((_HROWS, 1), lambda i: (i, 0)),
      out_shape=jax.ShapeDtypeStruct((LPAD, 1), jnp.float32),
  )(hs, hd, wp, bp)


def _pad_edges(ei, pad_to):
  src = ei[0].astype(jnp.int32)
  dst = ei[1].astype(jnp.int32)
  pad = pad_to - src.shape[0]
  src = jnp.concatenate([src, jnp.zeros((pad,), jnp.int32)])
  dst = jnp.concatenate([dst, jnp.full((pad,), NPAD - 1, jnp.int32)])
  return src, dst


def kernel(x_user, x_item, edge_index_buys, edge_index_rev, edge_label_index,
           snap, W1_buys, W1_rev, Wp1, bp1, a1_Wu, a1_bu, a1_qu, a1_Wi, a1_bi,
           a1_qi, W2_buys, W2_rev, Wp2, bp2, a2_Wu, a2_bu, a2_qu, a2_Wi,
           a2_bi, a2_qi, W_post, b_post):
  eb_src, eb_dst = _pad_edges(edge_index_buys, EPAD)
  er_src, er_dst = _pad_edges(edge_index_rev, EPAD)
  z64 = jnp.zeros((NPAD, HD), jnp.float32)
  z16 = jnp.zeros((NPAD, 16), jnp.float32)
  ones_h = jnp.ones((CHUNK, 16), jnp.float32)

  si_lo, si_hi, su_lo, su_hi, cnt_i, cnt_u = _seg_sum_sc(True)(
      x_user[:, :HD], x_user[:, HD:], x_item[:, :HD], x_item[:, HD:],
      eb_src, eb_dst, er_src, er_dst, z64, z16, ones_h)
  sums_i1 = jnp.concatenate([si_lo, si_hi], axis=1)
  sums_u1 = jnp.concatenate([su_lo, su_hi], axis=1)
  m_item1, m_user1 = _tc_dense(sums_i1, cnt_i, sums_u1, cnt_u,
                               W1_buys, W1_rev, Wp1, bp1.reshape(1, D))

  si_lo2, si_hi2, su_lo2, su_hi2 = _seg_sum_sc(False)(
      m_user1[:, :HD], m_user1[:, HD:], m_item1[:, :HD], m_item1[:, HD:],
      eb_src, eb_dst, er_src, er_dst, z64, z16, ones_h)
  sums_i2 = jnp.concatenate([si_lo2, si_hi2], axis=1)
  sums_u2 = jnp.concatenate([su_lo2, su_hi2], axis=1)
  m_item2, m_user2 = _tc_dense(sums_i2, cnt_i, sums_u2, cnt_u,
                               W2_buys, W2_rev, Wp2, bp2.reshape(1, D))

  lpad = LPAD - L
  ls = jnp.concatenate([edge_label_index[0].astype(jnp.int32),
                        jnp.zeros((lpad,), jnp.int32)])
  ld = jnp.concatenate([edge_label_index[1].astype(jnp.int32),
                        jnp.zeros((lpad,), jnp.int32)])
  hs, hd = _head_gather_sc()(m_user2, m_item2, ls, ld)
  h = _tc_head(hs, hd, W_post, b_post.reshape(1, 2))[:L, 0]

  return (h, m_user1[:N], m_item1[:N], m_user2[:N], m_item2[:N])
